# baseline (speedup 1.0000x reference)
import jax
import jax.numpy as jnp
from jax import lax
from jax.experimental import pallas as pl
from jax.experimental.pallas import tpu as pltpu
from jax.experimental.pallas import tpu_sc as plsc

_BS = 1024
_CR = 32
_RING = 3
_B_TC = 3


def _tc_copy_kernel(pos_ref, out_ref):
    blk = pos_ref[...]
    out_ref[...] = jnp.broadcast_to(blk[None, :, :], out_ref.shape)


def _make_sc_kernel(B, S, H, dtype):
    info = plsc.get_sparse_core_info()
    NC, NS = info.num_cores, info.num_subcores
    NW = NC * NS
    rows_per_w = S // NW
    nchunk = rows_per_w // _CR
    mesh = plsc.VectorSubcoreMesh(core_axis_name="c", subcore_axis_name="s")

    def body(pos_hbm, out_hbm, vbuf, in_sem, out_sem):
        wid = lax.axis_index("s") * NC + lax.axis_index("c")
        base = wid * rows_per_w

        def in_cp(k, slot):
            return pltpu.make_async_copy(
                pos_hbm.at[pl.ds(base + k * _CR, _CR), :],
                vbuf.at[slot],
                in_sem.at[slot],
            )

        def out_cp(k, slot, b):
            return pltpu.make_async_copy(
                vbuf.at[slot],
                out_hbm.at[b, pl.ds(base + k * _CR, _CR), :],
                out_sem.at[slot],
            )

        for k in range(_RING - 1):
            in_cp(k, k % _RING).start()
        for k in range(nchunk):
            slot = k % _RING
            in_cp(k, slot).wait()
            nxt = k + _RING - 1
            if nxt < nchunk:
                nslot = nxt % _RING
                if k >= 1:
                    for b in range(B):
                        out_cp(k - 1, nslot, b).wait()
                in_cp(nxt, nslot).start()
            for b in range(B):
                out_cp(k, slot, b).start()
        for k in range(max(0, nchunk - _RING), nchunk):
            for b in range(B):
                out_cp(k, k % _RING, b).wait()

    return pl.kernel(
        body,
        out_type=jax.ShapeDtypeStruct((B, S, H), dtype),
        mesh=mesh,
        scratch_types=[
            pltpu.VMEM((_RING, _CR, H), dtype),
            pltpu.SemaphoreType.DMA((_RING,)),
            pltpu.SemaphoreType.DMA((_RING,)),
        ],
    )


def kernel(x, pos_emb):
    B, S = x.shape
    N, H = pos_emb.shape
    b_sc = B - _B_TC
    tc_out = pl.pallas_call(
        _tc_copy_kernel,
        grid=(S // _BS,),
        in_specs=[pl.BlockSpec((_BS, H), lambda j: (j, 0))],
        out_specs=pl.BlockSpec((_B_TC, _BS, H), lambda j: (0, j, 0)),
        out_shape=jax.ShapeDtypeStruct((_B_TC, S, H), pos_emb.dtype),
    )(pos_emb)
    sc_out = _make_sc_kernel(b_sc, S, H, pos_emb.dtype)(pos_emb)
    return jnp.concatenate([tc_out, sc_out], axis=0)


# speedup vs baseline: 2.2194x; 2.2194x over previous
import jax
import jax.numpy as jnp
from jax import lax
from jax.experimental import pallas as pl
from jax.experimental.pallas import tpu as pltpu
from jax.experimental.pallas import tpu_sc as plsc

_SLOT_ROWS = (64, 56)


def _chunk_schedule(rows_per_w):
    sched = []
    off = 0
    i = 0
    while off < rows_per_w:
        slot = i % 2
        sz = min(_SLOT_ROWS[slot], rows_per_w - off)
        sched.append((off, sz, slot))
        off += sz
        i += 1
    return sched


def _make_sc_kernel(B, S, H, dtype):
    info = plsc.get_sparse_core_info()
    NC, NS = info.num_cores, info.num_subcores
    NW = NC * NS
    rows_per_w = S // NW
    sched = _chunk_schedule(rows_per_w)
    n = len(sched)
    mesh = plsc.VectorSubcoreMesh(core_axis_name="c", subcore_axis_name="s")

    def body(pos_hbm, out_hbm, vbuf_a, vbuf_b, in_sem, out_sem):
        bufs = (vbuf_a, vbuf_b)
        wid = lax.axis_index("s") * NC + lax.axis_index("c")
        base = wid * rows_per_w

        def in_cp(i):
            off, sz, slot = sched[i]
            return pltpu.make_async_copy(
                pos_hbm.at[pl.ds(base + off, sz), :],
                bufs[slot].at[pl.ds(0, sz), :],
                in_sem.at[slot],
            )

        def out_cp(i, b):
            off, sz, slot = sched[i]
            return pltpu.make_async_copy(
                bufs[slot].at[pl.ds(0, sz), :],
                out_hbm.at[b, pl.ds(base + off, sz), :],
                out_sem.at[slot],
            )

        in_cp(0).start()
        for i in range(n):
            in_cp(i).wait()
            if i + 1 < n:
                if i >= 1:
                    for b in range(B):
                        out_cp(i - 1, b).wait()
                in_cp(i + 1).start()
            for b in range(B):
                out_cp(i, b).start()
        for i in (n - 2, n - 1):
            for b in range(B):
                out_cp(i, b).wait()

    return pl.kernel(
        body,
        out_type=jax.ShapeDtypeStruct((B, S, H), dtype),
        mesh=mesh,
        scratch_types=[
            pltpu.VMEM((_SLOT_ROWS[0], H), dtype),
            pltpu.VMEM((_SLOT_ROWS[1], H), dtype),
            pltpu.SemaphoreType.DMA((2,)),
            pltpu.SemaphoreType.DMA((2,)),
        ],
    )


def kernel(x, pos_emb):
    B, S = x.shape
    N, H = pos_emb.shape
    return _make_sc_kernel(B, S, H, pos_emb.dtype)(pos_emb)
